# Initial kernel scaffold; baseline (speedup 1.0000x reference)
import jax
import jax.numpy as jnp
from jax.experimental import pallas as pl


def kernel(idx, v):
    raise NotImplementedError("write your pallas kernel here")


# speedup vs baseline: 1.2721x; 1.2721x over previous
import functools

import jax
import jax.numpy as jnp
from jax import lax
from jax.experimental import pallas as pl
from jax.experimental.pallas import tpu as pltpu
from jax.experimental.pallas import tpu_sc as plsc

B, K = 16384, 100
TOTAL = B * K
NW = 32
PER_W = TOTAL // NW


@jax.jit
def _sc_gather(v, idx_flat):
    mesh = plsc.VectorSubcoreMesh(core_axis_name="c", subcore_axis_name="s")

    @functools.partial(
        pl.kernel,
        mesh=mesh,
        out_type=jax.ShapeDtypeStruct((TOTAL,), jnp.float32),
        scratch_types=[
            pltpu.VMEM((PER_W,), jnp.int32),
            pltpu.VMEM((PER_W,), jnp.float32),
            pltpu.SemaphoreType.DMA,
        ],
    )
    def k(v_hbm, idx_hbm, out_hbm, idx_v, out_v, sem):
        wid = lax.axis_index("s") * 2 + lax.axis_index("c")
        base = wid * PER_W
        pltpu.sync_copy(idx_hbm.at[pl.ds(base, PER_W)], idx_v)
        pltpu.async_copy(v_hbm.at[idx_v], out_v, sem).wait()
        pltpu.sync_copy(out_v, out_hbm.at[pl.ds(base, PER_W)])

    return k(v, idx_flat)


def kernel(idx, v):
    out = _sc_gather(v, idx.reshape(TOTAL).astype(jnp.int32))
    return out.reshape(B, K)


# speedup vs baseline: 1.6867x; 1.3259x over previous
import functools

import jax
import jax.numpy as jnp
from jax import lax
from jax.experimental import pallas as pl
from jax.experimental.pallas import tpu as pltpu
from jax.experimental.pallas import tpu_sc as plsc

B, K = 16384, 100
NW = 32
ROWS_W = B // NW


@jax.jit
def _sc_gather(v, idx):
    mesh = plsc.VectorSubcoreMesh(core_axis_name="c", subcore_axis_name="s")

    @functools.partial(
        pl.kernel,
        mesh=mesh,
        out_type=jax.ShapeDtypeStruct((B, K), jnp.float32),
        scratch_types=[
            pltpu.VMEM((ROWS_W, K), jnp.int32),
            pltpu.VMEM((ROWS_W, K), jnp.float32),
            pltpu.SemaphoreType.DMA,
        ],
    )
    def k(v_hbm, idx_hbm, out_hbm, idx_v, out_v, sem):
        wid = lax.axis_index("s") * 2 + lax.axis_index("c")
        base = wid * ROWS_W
        pltpu.sync_copy(idx_hbm.at[pl.ds(base, ROWS_W)], idx_v)

        @pl.loop(0, ROWS_W)
        def _fire(r):
            pltpu.async_copy(v_hbm.at[idx_v.at[r]], out_v.at[r], sem)

        @pl.loop(0, ROWS_W)
        def _drain(r):
            pltpu.make_async_copy(v_hbm.at[idx_v.at[r]], out_v.at[r], sem).wait()

        pltpu.sync_copy(out_v, out_hbm.at[pl.ds(base, ROWS_W)])

    return k(v, idx)


def kernel(idx, v):
    return _sc_gather(v, idx.astype(jnp.int32))


# speedup vs baseline: 2.4869x; 1.4745x over previous
import functools

import jax
import jax.numpy as jnp
from jax import lax
from jax.experimental import pallas as pl
from jax.experimental.pallas import tpu as pltpu
from jax.experimental.pallas import tpu_sc as plsc

B, K = 16384, 100
NW = 32
ROWS_W = B // NW


@jax.jit
def _sc_gather(v, idx):
    mesh = plsc.VectorSubcoreMesh(core_axis_name="c", subcore_axis_name="s")

    @functools.partial(
        pl.kernel,
        mesh=mesh,
        out_type=jax.ShapeDtypeStruct((B, K), jnp.float32),
        scratch_types=[
            pltpu.VMEM_SHARED((1000000,), jnp.float32),
            pltpu.VMEM((128, K), jnp.int32),
            pltpu.VMEM((128, K), jnp.float32),
            pltpu.VMEM((20000,), jnp.float32),
            pltpu.SemaphoreType.DMA,
        ],
    )
    def k(v_hbm, idx_hbm, out_hbm, vs, idx_v, out_v, tmp, sem):
        sid = lax.axis_index("s")
        wid = sid * 2 + lax.axis_index("c")
        base = wid * ROWS_W

        @pl.loop(sid, 50, step=16)
        def _stage(c):
            off = c * 20000
            pltpu.sync_copy(v_hbm.at[pl.ds(off, 20000)], tmp)
            pltpu.sync_copy(tmp, vs.at[pl.ds(off, 20000)])

        plsc.subcore_barrier()

        @pl.loop(0, 4)
        def _round(h):
            row0 = base + h * 128
            pltpu.sync_copy(idx_hbm.at[pl.ds(row0, 128)], idx_v)

            @pl.loop(0, 128)
            def _fire(r):
                pltpu.async_copy(vs.at[idx_v.at[r]], out_v.at[r], sem)

            @pl.loop(0, 128)
            def _drain(r):
                pltpu.make_async_copy(vs.at[idx_v.at[r]], out_v.at[r], sem).wait()

            pltpu.sync_copy(out_v, out_hbm.at[pl.ds(row0, 128)])

    return k(v, idx)


def kernel(idx, v):
    return _sc_gather(v, idx.astype(jnp.int32))
